# baseline (speedup 1.0000x reference)
import jax
import jax.numpy as jnp
from jax.experimental import pallas as pl
from jax.experimental.pallas import tpu as pltpu

IN_CH, N_HID, N_OUT = 32, 64, 64
ALPHA = 0.1
NV, NE, NF = 4096, 12288, 8192

BME = 256
BMV = 128
NEB = NE // BME
NVB = NV // BMV
NBUF = 3
N_CELLS = NV + NE + NF
_VMEM_LIMIT = 65024 * 1024


def _leaky(x):
    return jnp.where(x >= 0, x, ALPHA * x)


def _bf(x):
    return x.astype(jnp.bfloat16)


def _cxn_kernel(gf2e_hbm, ge2v_hbm, xv_ref, xe_ref, xf_ref,
                wvT_ref, weT_ref, wfT_ref, we2vT_ref, wf2eT_ref,
                bv_ref, be_ref, bf_ref, be2v_ref, bf2e_ref,
                wlinT_ref, blin_ref, out_ref,
                ebuf, vbuf, acc_ref, esem, vsem):

    def ecopy(idx, slot):
        return pltpu.make_async_copy(
            gf2e_hbm.at[pl.ds(idx * BME, BME), :], ebuf.at[slot],
            esem.at[slot])

    def vcopy(idx, slot):
        return pltpu.make_async_copy(
            ge2v_hbm.at[pl.ds(idx * BMV, BMV), :], vbuf.at[slot],
            vsem.at[slot])

    for b in range(NBUF):
        ecopy(b, b).start()
    for b in range(NBUF):
        vcopy(b, b).start()

    pre_f = jnp.dot(xf_ref[...], wfT_ref[...],
                    preferred_element_type=jnp.float32) + bf_ref[...]
    acc_ref[...] = jnp.sum(_leaky(pre_f), axis=0, keepdims=True)

    def e_step(i, carry):
        slot = jax.lax.rem(i, NBUF)
        ecopy(i, slot).wait()
        m = jnp.dot(ebuf[slot], xf_ref[...],
                    preferred_element_type=jnp.float32)
        xe_blk = xe_ref[pl.ds(i * BME, BME), :]
        pre = (jnp.dot(xe_blk, weT_ref[...], preferred_element_type=jnp.float32)
               + jnp.dot(m, wf2eT_ref[...], preferred_element_type=jnp.float32)
               + be_ref[...] + bf2e_ref[...])
        acc_ref[...] += jnp.sum(_leaky(pre), axis=0, keepdims=True)

        @pl.when(i + NBUF < NEB)
        def _():
            ecopy(i + NBUF, slot).start()
        return carry

    jax.lax.fori_loop(0, NEB, e_step, 0)

    def v_step(j, carry):
        slot = jax.lax.rem(j, NBUF)
        vcopy(j, slot).wait()
        m = jnp.dot(vbuf[slot], xe_ref[...],
                    preferred_element_type=jnp.float32)
        xv_blk = xv_ref[pl.ds(j * BMV, BMV), :]
        pre = (jnp.dot(xv_blk, wvT_ref[...], preferred_element_type=jnp.float32)
               + jnp.dot(m, we2vT_ref[...], preferred_element_type=jnp.float32)
               + bv_ref[...] + be2v_ref[...])
        acc_ref[...] += jnp.sum(_leaky(pre), axis=0, keepdims=True)

        @pl.when(j + NBUF < NVB)
        def _():
            vcopy(j + NBUF, slot).start()
        return carry

    jax.lax.fori_loop(0, NVB, v_step, 0)

    z = jnp.maximum(acc_ref[...] * (1.0 / N_CELLS), 0.0)
    out_ref[...] = jnp.dot(z, wlinT_ref[...],
                           preferred_element_type=jnp.float32) + blin_ref[...]


@jax.jit
def kernel(xv, xe, xf, Ge2v, Gf2e, Wv, bv, We, be, Wf, bf,
           We2v, be2v, Wf2e, bf2e, Wlin, blin):
    row = lambda b: b.reshape(1, -1)
    vspec = pl.BlockSpec(memory_space=pltpu.MemorySpace.VMEM)
    aspec = pl.BlockSpec(memory_space=pltpu.MemorySpace.HBM)

    out = pl.pallas_call(
        _cxn_kernel,
        in_specs=[aspec, aspec] + [vspec] * 15,
        out_specs=vspec,
        out_shape=jax.ShapeDtypeStruct((1, N_OUT), jnp.float32),
        scratch_shapes=[
            pltpu.VMEM((NBUF, BME, NF), jnp.float32),
            pltpu.VMEM((NBUF, BMV, NE), jnp.float32),
            pltpu.VMEM((1, N_HID), jnp.float32),
            pltpu.SemaphoreType.DMA((NBUF,)),
            pltpu.SemaphoreType.DMA((NBUF,)),
        ],
        compiler_params=pltpu.CompilerParams(
            vmem_limit_bytes=_VMEM_LIMIT),
    )(Gf2e, Ge2v, xv[0], xe[0], xf[0],
      Wv.T, We.T, Wf.T, We2v.T, Wf2e.T,
      row(bv), row(be), row(bf), row(be2v), row(bf2e),
      Wlin.T, row(blin))
    return out
